# baseline (speedup 1.0000x reference)
import functools

import jax
import jax.numpy as jnp
from jax import lax
from jax.experimental import pallas as pl
from jax.experimental.pallas import tpu as pltpu
from jax.experimental.pallas import tpu_sc as plsc

B = 16
MAX_LEN = 4096
D = 1024
NW = 32
SLAB = 1024
NSLAB = (B * MAX_LEN) // SLAB
SPW = NSLAB // NW
ROWS_PER_W = SLAB * SPW
UNIT = 32
UPS = SLAB // UNIT
NUNITS = ROWS_PER_W // UNIT
ZROWS = 32
ZRING = 8
TOTAL_ROWS = B * MAX_LEN


def _build_kernel():
    mesh = plsc.VectorSubcoreMesh(core_axis_name="c", subcore_axis_name="s")

    def body(
        flat_hbm,
        params_hbm,
        zeros_hbm,
        out_hbm,
        pvec,
        zbuf,
        idx_all,
        vb0,
        vb1,
        vb2,
        gsem,
        ssem,
        zsem,
        lsem,
    ):
        wid = lax.axis_index("s") * 2 + lax.axis_index("c")

        pltpu.sync_copy(params_hbm.at[pl.ds(wid * 16, 16)], pvec)

        @pl.when(lax.axis_index("s") == 0)
        def _stage_zeros():
            pltpu.sync_copy(zeros_hbm, zbuf)

        plsc.subcore_barrier()

        pv = pvec[...]
        starts = (pv[0], pv[3])
        valids = (pv[1], pv[4])
        outbases = (
            pl.multiple_of(pv[2], UNIT),
            pl.multiple_of(pv[5], UNIT),
        )

        lane = lax.broadcasted_iota(jnp.int32, (16,), 0)

        for s in range(SPW):
            start_s = starts[s]

            def idx_body(i, carry, start_s=start_s, s=s):
                idx_all[pl.ds(s * SLAB + i * 16, 16)] = jnp.minimum(
                    start_s + i * 16 + lane, TOTAL_ROWS - 1
                )
                return carry

            lax.fori_loop(0, SLAB // 16, idx_body, 0)

        def advance(prev_pred, prev_gd, prev_sd):
            @pl.when(prev_pred)
            def _():
                prev_gd.wait()
                prev_sd.start()

        bufs = (vb0, vb1, vb2)
        units = []

        for u in range(NUNITS):
            s = u // UPS
            j = u % UPS
            is_copy = valids[s] >= (j + 1) * UNIT
            buf = bufs[u % 3]
            dst = outbases[s] + j * UNIT
            gd = pltpu.make_async_copy(
                flat_hbm.at[idx_all.at[pl.ds(u * UNIT, UNIT)]], buf, gsem
            )
            sd = pltpu.make_async_copy(buf, out_hbm.at[pl.ds(dst, UNIT)], ssem)
            zda = pltpu.make_async_copy(
                zbuf, out_hbm.at[pl.ds(dst, UNIT)], zsem
            )

            if u >= 3:
                pred3, _, sd3, _ = units[u - 3]

                @pl.when(pred3)
                def _wait_scatter(sd3=sd3):
                    sd3.wait()

            @pl.when(is_copy)
            def _start_gather(gd=gd):
                gd.start()

            @pl.when(jnp.logical_not(is_copy))
            def _start_zero(zda=zda):
                zda.start()

            if u >= 1:
                pu = units[u - 1]
                advance(pu[0], pu[1], pu[2])

            units.append((is_copy, gd, sd, zda))

            if u >= ZRING:
                predz, _, _, za = units[u - ZRING]

                @pl.when(jnp.logical_not(predz))
                def _wait_zero(za=za):
                    za.wait()

        pu = units[NUNITS - 1]
        advance(pu[0], pu[1], pu[2])
        for u in (NUNITS - 3, NUNITS - 2, NUNITS - 1):
            predu, _, sdu, _ = units[u]

            @pl.when(predu)
            def _wait_scatter_tail(sdu=sdu):
                sdu.wait()

        for u in range(NUNITS - ZRING, NUNITS):
            predu, _, _, za = units[u]

            @pl.when(jnp.logical_not(predu))
            def _wait_zero_tail(za=za):
                za.wait()

        u0s = tuple(valids[s] // UNIT for s in range(SPW))
        ps = tuple(valids[s] - u0s[s] * UNIT for s in range(SPW))
        sgd = []
        for s in range(SPW):
            gd = pltpu.make_async_copy(
                flat_hbm.at[
                    idx_all.at[
                        pl.ds(
                            pl.multiple_of(s * SLAB + u0s[s] * UNIT, UNIT),
                            UNIT,
                        )
                    ]
                ],
                bufs[s],
                gsem,
            )

            @pl.when(ps[s] > 0)
            def _start_straddle_gather(gd=gd):
                gd.start()

            sgd.append(gd)

        ssd = []
        for s in range(SPW):
            sd = pltpu.make_async_copy(
                bufs[s],
                out_hbm.at[
                    pl.ds(
                        pl.multiple_of(outbases[s] + u0s[s] * UNIT, UNIT),
                        UNIT,
                    )
                ],
                ssem,
            )

            @pl.when(ps[s] > 0)
            def _patch_and_scatter(gd=sgd[s], sd=sd, s=s):
                gd.wait()
                zero16 = jnp.zeros((16,), jnp.float32)
                buf = bufs[s]
                p = ps[s]

                def zrow(i, carry):
                    r = p + i
                    for c in range(D // 16):
                        buf[r, pl.ds(c * 16, 16)] = zero16
                    return carry

                lax.fori_loop(0, UNIT - p, zrow, 0)
                sd.start()

            ssd.append(sd)

        for s in range(SPW):

            @pl.when(ps[s] > 0)
            def _wait_straddle_scatter(sd=ssd[s]):
                sd.wait()

    return functools.partial(
        pl.kernel,
        out_type=jax.ShapeDtypeStruct((B * MAX_LEN, D), jnp.float32),
        mesh=mesh,
        scratch_types=[
            pltpu.VMEM((16,), jnp.int32),
            pltpu.VMEM_SHARED((ZROWS, D), jnp.float32),
            pltpu.VMEM((ROWS_PER_W,), jnp.int32),
            pltpu.VMEM((UNIT, D), jnp.float32),
            pltpu.VMEM((UNIT, D), jnp.float32),
            pltpu.VMEM((UNIT, D), jnp.float32),
            pltpu.SemaphoreType.DMA,
            pltpu.SemaphoreType.DMA,
            pltpu.SemaphoreType.DMA,
            pltpu.SemaphoreType.DMA,
        ],
    )(body)


_pad_kernel = _build_kernel()


def kernel(flat, cu_seqlens):
    cu = cu_seqlens.astype(jnp.int32)
    lens32 = cu[1:] - cu[:-1]

    g = jnp.arange(NSLAB, dtype=jnp.int32)
    b = g // (MAX_LEN // SLAB)
    t0 = (g % (MAX_LEN // SLAB)) * SLAB
    sstart = cu[:-1][b] + t0
    svalid = jnp.clip(lens32[b] - t0, 0, SLAB)
    soutbase = g * SLAB

    order = jnp.argsort(-svalid)
    w = jnp.arange(NW, dtype=jnp.int32)
    g0 = order[w]
    g1 = order[NSLAB - 1 - w]
    params = jnp.zeros((NW, 16), jnp.int32)
    params = (
        params.at[:, 0].set(sstart[g0])
        .at[:, 1].set(svalid[g0])
        .at[:, 2].set(soutbase[g0])
        .at[:, 3].set(sstart[g1])
        .at[:, 4].set(svalid[g1])
        .at[:, 5].set(soutbase[g1])
    )

    zeros = jnp.zeros((ZROWS, D), jnp.float32)
    out = _pad_kernel(flat, params.reshape(-1), zeros)
    padded = out.reshape(B, MAX_LEN, D)
    lens = lens32.astype(jnp.int64)
    return padded, lens


# speedup vs baseline: 1.0735x; 1.0735x over previous
import functools

import jax
import jax.numpy as jnp
from jax import lax
from jax.experimental import pallas as pl
from jax.experimental.pallas import tpu as pltpu
from jax.experimental.pallas import tpu_sc as plsc

B = 16
MAX_LEN = 4096
D = 1024
NW = 32
SLAB = 1024
NSLAB = (B * MAX_LEN) // SLAB
SPW = NSLAB // NW
ROWS_PER_W = SLAB * SPW
UNIT = 32
UPS = SLAB // UNIT
NUNITS = ROWS_PER_W // UNIT
ZROWS = 16
ZRING = 8
TOTAL_ROWS = B * MAX_LEN


def _build_kernel():
    mesh = plsc.VectorSubcoreMesh(core_axis_name="c", subcore_axis_name="s")

    def body(
        flat_hbm,
        params_hbm,
        zeros_hbm,
        out_hbm,
        pvec,
        zbuf,
        idx_all,
        vb0,
        vb1,
        vb2,
        gsem,
        ssem,
        zsem,
        lsem,
    ):
        wid = lax.axis_index("s") * 2 + lax.axis_index("c")

        pltpu.sync_copy(params_hbm.at[pl.ds(wid * 16, 16)], pvec)
        pltpu.sync_copy(zeros_hbm, zbuf)

        pv = pvec[...]
        starts = (pv[0], pv[3])
        valids = (pv[1], pv[4])
        outbases = (
            pl.multiple_of(pv[2], UNIT),
            pl.multiple_of(pv[5], UNIT),
        )

        lane = lax.broadcasted_iota(jnp.int32, (16,), 0)

        for s in range(SPW):
            start_s = starts[s]

            def idx_body(i, carry, start_s=start_s, s=s):
                idx_all[pl.ds(s * SLAB + i * 16, 16)] = jnp.minimum(
                    start_s + i * 16 + lane, TOTAL_ROWS - 1
                )
                return carry

            lax.fori_loop(0, SLAB // 16, idx_body, 0)

        def advance(prev_pred, prev_gd, prev_sd):
            @pl.when(prev_pred)
            def _():
                prev_gd.wait()
                prev_sd.start()

        bufs = (vb0, vb1, vb2)
        units = []

        for u in range(NUNITS):
            s = u // UPS
            j = u % UPS
            is_copy = valids[s] >= (j + 1) * UNIT
            buf = bufs[u % 3]
            dst = outbases[s] + j * UNIT
            gd = pltpu.make_async_copy(
                flat_hbm.at[idx_all.at[pl.ds(u * UNIT, UNIT)]], buf, gsem
            )
            sd = pltpu.make_async_copy(buf, out_hbm.at[pl.ds(dst, UNIT)], ssem)
            zda = pltpu.make_async_copy(
                zbuf, out_hbm.at[pl.ds(dst, ZROWS)], zsem
            )
            zdb = pltpu.make_async_copy(
                zbuf, out_hbm.at[pl.ds(dst + ZROWS, ZROWS)], zsem
            )

            if u >= 3:
                pred3, _, sd3, _, _ = units[u - 3]

                @pl.when(pred3)
                def _wait_scatter(sd3=sd3):
                    sd3.wait()

            @pl.when(is_copy)
            def _start_gather(gd=gd):
                gd.start()

            @pl.when(jnp.logical_not(is_copy))
            def _start_zero(zda=zda, zdb=zdb):
                zda.start()
                zdb.start()

            if u >= 1:
                pu = units[u - 1]
                advance(pu[0], pu[1], pu[2])

            units.append((is_copy, gd, sd, zda, zdb))

            if u >= ZRING:
                predz, _, _, za, zb = units[u - ZRING]

                @pl.when(jnp.logical_not(predz))
                def _wait_zero(za=za, zb=zb):
                    za.wait()
                    zb.wait()

        pu = units[NUNITS - 1]
        advance(pu[0], pu[1], pu[2])
        for u in (NUNITS - 3, NUNITS - 2, NUNITS - 1):
            predu, _, sdu, _, _ = units[u]

            @pl.when(predu)
            def _wait_scatter_tail(sdu=sdu):
                sdu.wait()

        for u in range(NUNITS - ZRING, NUNITS):
            predu, _, _, za, zb = units[u]

            @pl.when(jnp.logical_not(predu))
            def _wait_zero_tail(za=za, zb=zb):
                za.wait()
                zb.wait()

        u0s = tuple(valids[s] // UNIT for s in range(SPW))
        ps = tuple(valids[s] - u0s[s] * UNIT for s in range(SPW))
        sgd = []
        for s in range(SPW):
            gd = pltpu.make_async_copy(
                flat_hbm.at[
                    idx_all.at[
                        pl.ds(
                            pl.multiple_of(s * SLAB + u0s[s] * UNIT, UNIT),
                            UNIT,
                        )
                    ]
                ],
                bufs[s],
                gsem,
            )

            @pl.when(ps[s] > 0)
            def _start_straddle_gather(gd=gd):
                gd.start()

            sgd.append(gd)

        ssd = []
        for s in range(SPW):
            sd = pltpu.make_async_copy(
                bufs[s],
                out_hbm.at[
                    pl.ds(
                        pl.multiple_of(outbases[s] + u0s[s] * UNIT, UNIT),
                        UNIT,
                    )
                ],
                ssem,
            )

            @pl.when(ps[s] > 0)
            def _patch_and_scatter(gd=sgd[s], sd=sd, s=s):
                gd.wait()
                zero16 = jnp.zeros((16,), jnp.float32)
                buf = bufs[s]
                p = ps[s]

                def zrow(i, carry):
                    r = p + i
                    for c in range(D // 16):
                        buf[r, pl.ds(c * 16, 16)] = zero16
                    return carry

                lax.fori_loop(0, UNIT - p, zrow, 0)
                sd.start()

            ssd.append(sd)

        for s in range(SPW):

            @pl.when(ps[s] > 0)
            def _wait_straddle_scatter(sd=ssd[s]):
                sd.wait()

    return functools.partial(
        pl.kernel,
        out_type=jax.ShapeDtypeStruct((B * MAX_LEN, D), jnp.float32),
        mesh=mesh,
        scratch_types=[
            pltpu.VMEM((16,), jnp.int32),
            pltpu.VMEM((ZROWS, D), jnp.float32),
            pltpu.VMEM((ROWS_PER_W,), jnp.int32),
            pltpu.VMEM((UNIT, D), jnp.float32),
            pltpu.VMEM((UNIT, D), jnp.float32),
            pltpu.VMEM((UNIT, D), jnp.float32),
            pltpu.SemaphoreType.DMA,
            pltpu.SemaphoreType.DMA,
            pltpu.SemaphoreType.DMA,
            pltpu.SemaphoreType.DMA,
        ],
    )(body)


_pad_kernel = _build_kernel()


def kernel(flat, cu_seqlens):
    cu = cu_seqlens.astype(jnp.int32)
    lens32 = cu[1:] - cu[:-1]

    g = jnp.arange(NSLAB, dtype=jnp.int32)
    b = g // (MAX_LEN // SLAB)
    t0 = (g % (MAX_LEN // SLAB)) * SLAB
    sstart = cu[:-1][b] + t0
    svalid = jnp.clip(lens32[b] - t0, 0, SLAB)
    soutbase = g * SLAB

    order = jnp.argsort(-svalid)
    w = jnp.arange(NW, dtype=jnp.int32)
    g0 = order[w]
    g1 = order[NSLAB - 1 - w]
    params = jnp.zeros((NW, 16), jnp.int32)
    params = (
        params.at[:, 0].set(sstart[g0])
        .at[:, 1].set(svalid[g0])
        .at[:, 2].set(soutbase[g0])
        .at[:, 3].set(sstart[g1])
        .at[:, 4].set(svalid[g1])
        .at[:, 5].set(soutbase[g1])
    )

    zeros = jnp.zeros((ZROWS, D), jnp.float32)
    out = _pad_kernel(flat, params.reshape(-1), zeros)
    padded = out.reshape(B, MAX_LEN, D)
    lens = lens32.astype(jnp.int64)
    return padded, lens
